# baseline (speedup 1.0000x reference)
import functools

import jax
import jax.numpy as jnp
from jax import lax
from jax.experimental import pallas as pl
from jax.experimental.pallas import tpu as pltpu
from jax.experimental.pallas import tpu_sc as plsc

_S = 512
_N = 4096
_B = 8
_RADII = (0.1, 0.2, 0.4)
_KS = (16, 32, 64)
_FPAD = 32
_SENT = _N
_GRP = 8


def _fps_body(x_ref, y_ref, z_ref, cx_ref, cy_ref, cz_ref, cn2_ref, xn2_ref):
    x = x_ref[...]
    y = y_ref[...]
    z = z_ref[...]
    xn2_ref[...] = x * x + y * y + z * z
    B, N = x.shape
    lane_n = lax.broadcasted_iota(jnp.int32, (B, N), 1)
    lane_s = lax.broadcasted_iota(jnp.int32, (B, _S), 1)

    def body(i, carry):
        dist, far, cxs, cys, czs = carry
        oh = (lane_n == far).astype(jnp.float32)
        cx = jnp.sum(x * oh, axis=1, keepdims=True)
        cy = jnp.sum(y * oh, axis=1, keepdims=True)
        cz = jnp.sum(z * oh, axis=1, keepdims=True)
        sel = lane_s == i
        cxs = jnp.where(sel, cx, cxs)
        cys = jnp.where(sel, cy, cys)
        czs = jnp.where(sel, cz, czs)
        def sq(v):
            s = lax.bitcast_convert_type(v * v, jnp.int32)
            return lax.bitcast_convert_type(s, jnp.float32)

        d = (sq(x - cx) + sq(y - cy)) + sq(z - cz)
        dist = jnp.minimum(dist, d)
        m = jnp.max(dist, axis=1, keepdims=True)
        far = jnp.min(jnp.where(dist == m, lane_n, N), axis=1, keepdims=True)
        return dist, far, cxs, cys, czs

    dist0 = jnp.full((B, N), 1e10, jnp.float32)
    far0 = jnp.zeros((B, 1), jnp.int32)
    zs = jnp.zeros((B, _S), jnp.float32)
    _, _, cxs, cys, czs = lax.fori_loop(0, _S, body, (dist0, far0, zs, zs, zs))
    cx_ref[...] = cxs
    cy_ref[...] = cys
    cz_ref[...] = czs
    cn2_ref[...] = cxs * cxs + cys * cys + czs * czs


def _fps(x, y, z):
    B, N = x.shape
    f32 = jnp.float32
    return pl.pallas_call(
        _fps_body,
        out_shape=(
            jax.ShapeDtypeStruct((B, _S), f32),
            jax.ShapeDtypeStruct((B, _S), f32),
            jax.ShapeDtypeStruct((B, _S), f32),
            jax.ShapeDtypeStruct((B, _S), f32),
            jax.ShapeDtypeStruct((B, N), f32),
        ),
    )(x, y, z)


def _bf16_rne(v):
    u = plsc.bitcast(v, jnp.int32)
    u = (u + 0x7FFF + ((u >> 16) & 1)) & jnp.int32(-65536)
    return plsc.bitcast(u, jnp.float32)


def _sc_body(half, x_hbm, y_hbm, z_hbm, xn2_hbm, cx_hbm, cy_hbm, cz_hbm,
             cn2_hbm, ftab_hbm, g1_hbm, g2_hbm, g3_hbm,
             xv, yv, zv, nv, cxv, cyv, czv, c2v, jb1, jb2, jb3,
             ib1, ib2, ib3, r1, r2, r3, semg0, semg1, sems0, sems1):
    wid = lax.axis_index("s") * 2 + lax.axis_index("c")
    b = wid // 4
    slot = wid % 4
    _SW = _S // 2 // 4
    s0 = half * (_S // 2) + slot * _SW
    o0 = slot * _SW

    pltpu.sync_copy(x_hbm.at[b], xv)
    pltpu.sync_copy(y_hbm.at[b], yv)
    pltpu.sync_copy(z_hbm.at[b], zv)
    pltpu.sync_copy(xn2_hbm.at[b], nv)
    pltpu.sync_copy(cx_hbm.at[b, pl.ds(s0, _SW)], cxv)
    pltpu.sync_copy(cy_hbm.at[b, pl.ds(s0, _SW)], cyv)
    pltpu.sync_copy(cz_hbm.at[b, pl.ds(s0, _SW)], czv)
    pltpu.sync_copy(cn2_hbm.at[b, pl.ds(s0, _SW)], c2v)

    def round_buf(j, _):
        xv[pl.ds(j * 16, 16)] = -2.0 * _bf16_rne(xv[pl.ds(j * 16, 16)])
        yv[pl.ds(j * 16, 16)] = -2.0 * _bf16_rne(yv[pl.ds(j * 16, 16)])
        zv[pl.ds(j * 16, 16)] = -2.0 * _bf16_rne(zv[pl.ds(j * 16, 16)])
        return _

    lax.fori_loop(0, _N // 16, round_buf, 0)

    iota16 = lax.broadcasted_iota(jnp.int32, (16,), 0)
    rsq = tuple(jnp.float32(r * r) for r in _RADII)
    kvec = tuple(jnp.full((16,), k, jnp.int32) for k in _KS)
    zvec = jnp.zeros((16,), jnp.int32)

    def scan_group(grp, ibs):
        def per_centroid(gi, carry2):
            si = grp * _GRP + gi
            t = (si // 16) * 16
            lm = iota16 == si % 16
            cxs = jnp.sum(jnp.where(lm, _bf16_rne(cxv[pl.ds(t, 16)]), 0.0))
            cys = jnp.sum(jnp.where(lm, _bf16_rne(cyv[pl.ds(t, 16)]), 0.0))
            czs = jnp.sum(jnp.where(lm, _bf16_rne(czv[pl.ds(t, 16)]), 0.0))
            c2s = jnp.sum(jnp.where(lm, c2v[pl.ds(t, 16)], 0.0))

            def chunk4(j4, cnts):
                c1v_, c2v_, c3v_ = cnts
                for u in range(4):
                    base = j4 * 64 + u * 16
                    xc = xv[pl.ds(base, 16)]
                    yc = yv[pl.ds(base, 16)]
                    zc = zv[pl.ds(base, 16)]
                    nc = nv[pl.ds(base, 16)]
                    dist = (c2s + nc) + (cxs * xc + cys * yc + czs * zc)
                    idxv = base + iota16
                    m1 = dist <= rsq[0]
                    m2 = dist <= rsq[1]
                    m3 = dist <= rsq[2]
                    plsc.store_compressed(jb1.at[pl.ds(c1v_[0], 16)], idxv,
                                          mask=m1)
                    plsc.store_compressed(jb2.at[pl.ds(c2v_[0], 16)], idxv,
                                          mask=m2)
                    plsc.store_compressed(jb3.at[pl.ds(c3v_[0], 16)], idxv,
                                          mask=m3)
                    c1v_ = jnp.minimum(
                        c1v_ + plsc.all_reduce_population_count(m1), kvec[0])
                    c2v_ = jnp.minimum(
                        c2v_ + plsc.all_reduce_population_count(m2), kvec[1])
                    c3v_ = jnp.minimum(
                        c3v_ + plsc.all_reduce_population_count(m3), kvec[2])
                return c1v_, c2v_, c3v_

            c1v_, c2v_, c3v_ = lax.fori_loop(0, _N // 64, chunk4,
                                             (zvec, zvec, zvec))
            off = b * _N

            def finalize(jb, ib, K, cnt):
                v0 = jb[pl.ds(0, 16)]
                first = jnp.where(cnt > 0, v0[0], _N - 1)
                for t in range(K // 16):
                    v = jb[pl.ds(16 * t, 16)] if t else v0
                    m = (16 * t + iota16) < cnt
                    ib[pl.ds(gi * K + 16 * t, 16)] = (
                        jnp.where(m, v, first) + off)

            finalize(jb1, ibs[0], _KS[0], c1v_[0])
            finalize(jb2, ibs[1], _KS[1], c2v_[0])
            finalize(jb3, ibs[2], _KS[2], c3v_[0])
            return carry2

        lax.fori_loop(0, _GRP, per_centroid, 0)

    ibs = ((ib1.at[0], ib2.at[0], ib3.at[0]),
           (ib1.at[1], ib2.at[1], ib3.at[1]))
    rws = ((r1.at[0], r2.at[0], r3.at[0]),
           (r1.at[1], r2.at[1], r3.at[1]))
    semg = (semg0, semg1)
    sems = (sems0, sems1)
    ghbm = (g1_hbm, g2_hbm, g3_hbm)

    def gather_descs(par):
        return [pltpu.make_async_copy(ftab_hbm.at[ibs[par][k]],
                                      rws[par][k], semg[par])
                for k in range(3)]

    def store_descs(par, grp):
        gs0 = b * (_S // 2) + o0 + grp * _GRP
        return [pltpu.make_async_copy(
            rws[par][k],
            ghbm[k].at[pl.ds(gs0 * _KS[k], _GRP * _KS[k])], sems[par])
            for k in range(3)]

    def issue(descs):
        for d in descs:
            d.start()

    def wait(descs):
        for d in descs:
            d.wait()

    def slot(grp, par, guarded):
        scan_group(grp, ibs[par])
        other = 1 - par
        if guarded:
            wait(gather_descs(other))
            issue(store_descs(other, grp - 1))
            pl.when(grp >= 2)(lambda: wait(store_descs(par, grp - 2)))
        issue(gather_descs(par))

    def pair(g2, carry):
        grp_e = 2 * g2

        @pl.when(g2 == 0)
        def _():
            slot(0, 0, False)

        @pl.when(g2 > 0)
        def _():
            slot(grp_e, 0, True)

        slot(grp_e + 1, 1, True)
        return carry

    npair = _SW // _GRP // 2
    lax.fori_loop(0, npair, pair, 0)
    last = _SW // _GRP - 1
    wait(gather_descs(1))
    issue(store_descs(1, last))
    wait(store_descs(0, last - 1))
    wait(store_descs(1, last))


def _sc_group_gather(x, y, z, xn2, cx, cy, cz, cn2, ftab, half):
    f32 = jnp.float32
    i32 = jnp.int32
    _SW = _S // 2 // 4
    mesh = plsc.VectorSubcoreMesh(core_axis_name="c", subcore_axis_name="s")
    fn = pl.kernel(
        functools.partial(_sc_body, half),
        out_type=(
            jax.ShapeDtypeStruct((_B * (_S // 2) * _KS[0], _FPAD),
                                 jnp.bfloat16),
            jax.ShapeDtypeStruct((_B * (_S // 2) * _KS[1], _FPAD),
                                 jnp.bfloat16),
            jax.ShapeDtypeStruct((_B * (_S // 2) * _KS[2], _FPAD),
                                 jnp.bfloat16),
        ),
        mesh=mesh,
        compiler_params=pltpu.CompilerParams(
            needs_layout_passes=False, use_tc_tiling_on_sc=False),
        scratch_types=[
            pltpu.VMEM((_N,), f32),
            pltpu.VMEM((_N,), f32),
            pltpu.VMEM((_N,), f32),
            pltpu.VMEM((_N,), f32),
            pltpu.VMEM((_SW,), f32),
            pltpu.VMEM((_SW,), f32),
            pltpu.VMEM((_SW,), f32),
            pltpu.VMEM((_SW,), f32),
            pltpu.VMEM((_KS[0] + 16,), i32),
            pltpu.VMEM((_KS[1] + 16,), i32),
            pltpu.VMEM((_KS[2] + 16,), i32),
            pltpu.VMEM((2, _GRP * _KS[0]), i32),
            pltpu.VMEM((2, _GRP * _KS[1]), i32),
            pltpu.VMEM((2, _GRP * _KS[2]), i32),
            pltpu.VMEM((2, _GRP * _KS[0], _FPAD), jnp.bfloat16),
            pltpu.VMEM((2, _GRP * _KS[1], _FPAD), jnp.bfloat16),
            pltpu.VMEM((2, _GRP * _KS[2], _FPAD), jnp.bfloat16),
            pltpu.SemaphoreType.DMA,
            pltpu.SemaphoreType.DMA,
            pltpu.SemaphoreType.DMA,
            pltpu.SemaphoreType.DMA,
        ],
    )
    return fn(x, y, z, xn2, cx, cy, cz, cn2, ftab)


def _mlp_body(K, g_ref, c_ref, w1_ref, b1_ref, wc_ref, w2_ref, b2_ref,
              w3_ref, b3_ref, o_ref):
    bf16 = jnp.bfloat16
    X = g_ref[...]
    H = jnp.dot(X, w1_ref[...].astype(bf16),
                preferred_element_type=jnp.float32)
    O = jnp.dot(c_ref[...].astype(bf16), wc_ref[...].astype(bf16),
                preferred_element_type=jnp.float32)
    SB, C1 = O.shape
    H = H.reshape(SB, K, C1) - O[:, None, :]
    H = jnp.maximum(H + b1_ref[...][None, :, :], 0.0)
    H = H.reshape(SB * K, C1)
    H = jnp.maximum(
        jnp.dot(H.astype(bf16), w2_ref[...].astype(bf16),
                preferred_element_type=jnp.float32)
        + b2_ref[...], 0.0)
    H = jnp.maximum(
        jnp.dot(H.astype(bf16), w3_ref[...].astype(bf16),
                preferred_element_type=jnp.float32)
        + b3_ref[...], 0.0)
    C3 = H.shape[1]
    o_ref[...] = jnp.max(H.reshape(SB, K, C3), axis=1)


def _fold(branch):
    out = []
    for (W, b, gamma, beta) in branch:
        scale = gamma / jnp.sqrt(1.0 + 1e-3)
        out.append((W * scale[None, :], b * scale + beta))
    return out


def _mlp(K, g, c4, folded):
    (W1, b1), (W2, b2), (W3, b3) = folded
    C1, C2, C3 = W1.shape[1], W2.shape[1], W3.shape[1]
    W1p = jnp.zeros((_FPAD, C1), jnp.float32).at[:19, :].set(W1)
    Wc = jnp.zeros((4, C1), jnp.float32).at[:3, :].set(W1[16:19, :])
    SB = 128
    BS = g.shape[0] // K
    grid = (BS // SB,)
    return pl.pallas_call(
        functools.partial(_mlp_body, K),
        grid=grid,
        in_specs=[
            pl.BlockSpec((SB * K, _FPAD), lambda i: (i, 0)),
            pl.BlockSpec((SB, 4), lambda i: (i, 0)),
            pl.BlockSpec((_FPAD, C1), lambda i: (0, 0)),
            pl.BlockSpec((1, C1), lambda i: (0, 0)),
            pl.BlockSpec((4, C1), lambda i: (0, 0)),
            pl.BlockSpec((C1, C2), lambda i: (0, 0)),
            pl.BlockSpec((1, C2), lambda i: (0, 0)),
            pl.BlockSpec((C2, C3), lambda i: (0, 0)),
            pl.BlockSpec((1, C3), lambda i: (0, 0)),
        ],
        out_specs=pl.BlockSpec((SB, C3), lambda i: (i, 0)),
        out_shape=jax.ShapeDtypeStruct((BS, C3), jnp.float32),
    )(g, c4, W1p, b1[None, :], Wc, W2, b2[None, :], W3, b3[None, :])


def _fps_scan(xyz_t):
    B, N, C = xyz_t.shape

    def body(state, i):
        centroids, distance, farthest = state
        centroids = centroids.at[:, i].set(farthest)
        centroid = jnp.take_along_axis(xyz_t, farthest[:, None, None], axis=1)
        dist = jnp.sum((xyz_t - centroid) ** 2, -1)
        distance = jnp.minimum(distance, dist)
        farthest = jnp.argmax(distance, -1).astype(jnp.int32)
        return (centroids, distance, farthest), None

    centroids = jnp.zeros((B, _S), dtype=jnp.int32)
    distance = jnp.full((B, N), 1e10, dtype=jnp.float32)
    farthest = jnp.zeros((B,), dtype=jnp.int32)
    (centroids, _, _), _ = jax.lax.scan(
        body, (centroids, distance, farthest), jnp.arange(_S))
    return centroids


def kernel(xyz, points, params):
    B, _, N = xyz.shape
    x = xyz[:, 0, :]
    y = xyz[:, 1, :]
    z = xyz[:, 2, :]
    xyz_tt = jnp.transpose(xyz, (0, 2, 1))
    fps_idx = _fps_scan(xyz_tt)
    cxyz = jax.vmap(lambda p, i: p[i])(xyz_tt, fps_idx)
    cx = cxyz[:, :, 0]
    cy = cxyz[:, :, 1]
    cz = cxyz[:, :, 2]
    cn2 = jnp.sum(cxyz ** 2, -1)
    xn2 = jnp.sum(xyz_tt ** 2, -1)

    pts_t = jnp.transpose(points, (0, 2, 1))
    xyz_t = jnp.transpose(xyz, (0, 2, 1))
    ftab = jnp.concatenate(
        [pts_t, xyz_t, jnp.zeros((B, N, _FPAD - 19), jnp.float32)],
        axis=-1).reshape(B * N, _FPAD).astype(jnp.bfloat16)

    folded = [_fold(p) for p in params]
    halves = []
    for half in (0, 1):
        g1, g2, g3 = _sc_group_gather(x, y, z, xn2, cx, cy, cz, cn2, ftab,
                                      half)
        sl = slice(half * (_S // 2), (half + 1) * (_S // 2))
        c4 = jnp.stack(
            [cx[:, sl], cy[:, sl], cz[:, sl], jnp.zeros_like(cx[:, sl])],
            axis=-1).reshape(B * (_S // 2), 4)
        outs = [_mlp(_KS[bi], g, c4, folded[bi])
                for bi, g in enumerate((g1, g2, g3))]
        halves.append(
            jnp.concatenate(outs, axis=-1).reshape(B, _S // 2, 320))
    new_points = jnp.concatenate(halves, axis=1)
    new_points = jnp.transpose(new_points, (0, 2, 1))
    new_xyz = jnp.stack([cx, cy, cz], axis=1)
    return new_xyz, new_points


# speedup vs baseline: 8.0461x; 8.0461x over previous
import functools

import jax
import jax.numpy as jnp
from jax import lax
from jax.experimental import pallas as pl
from jax.experimental.pallas import tpu as pltpu
from jax.experimental.pallas import tpu_sc as plsc

_S = 512
_N = 4096
_B = 8
_RADII = (0.1, 0.2, 0.4)
_KS = (16, 32, 64)
_FPAD = 32
_SENT = _N
_GRP = 8


def _fps_body(x_ref, y_ref, z_ref, cx_ref, cy_ref, cz_ref, cn2_ref, xn2_ref):
    x = x_ref[...]
    y = y_ref[...]
    z = z_ref[...]
    xn2_ref[...] = x * x + y * y + z * z
    B, N = x.shape
    lane_n = lax.broadcasted_iota(jnp.int32, (B, N), 1)
    lane_s = lax.broadcasted_iota(jnp.int32, (B, _S), 1)

    def body(i, carry):
        dist, far, cxs, cys, czs = carry
        oh = (lane_n == far).astype(jnp.float32)
        cx = jnp.sum(x * oh, axis=1, keepdims=True)
        cy = jnp.sum(y * oh, axis=1, keepdims=True)
        cz = jnp.sum(z * oh, axis=1, keepdims=True)
        sel = lane_s == i
        cxs = jnp.where(sel, cx, cxs)
        cys = jnp.where(sel, cy, cys)
        czs = jnp.where(sel, cz, czs)
        dx = x - cx
        dy = y - cy
        dz = z - cz
        d = dx * dx + dy * dy + dz * dz
        dist = jnp.minimum(dist, d)
        m = jnp.max(dist, axis=1, keepdims=True)
        far = jnp.min(jnp.where(dist == m, lane_n, N), axis=1, keepdims=True)
        return dist, far, cxs, cys, czs

    dist0 = jnp.full((B, N), 1e10, jnp.float32)
    far0 = jnp.zeros((B, 1), jnp.int32)
    zs = jnp.zeros((B, _S), jnp.float32)
    _, _, cxs, cys, czs = lax.fori_loop(0, _S, body, (dist0, far0, zs, zs, zs))
    cx_ref[...] = cxs
    cy_ref[...] = cys
    cz_ref[...] = czs
    cn2_ref[...] = cxs * cxs + cys * cys + czs * czs


def _fps(x, y, z):
    B, N = x.shape
    f32 = jnp.float32
    return pl.pallas_call(
        _fps_body,
        out_shape=(
            jax.ShapeDtypeStruct((B, _S), f32),
            jax.ShapeDtypeStruct((B, _S), f32),
            jax.ShapeDtypeStruct((B, _S), f32),
            jax.ShapeDtypeStruct((B, _S), f32),
            jax.ShapeDtypeStruct((B, N), f32),
        ),
    )(x, y, z)


def _bf16_rne(v):
    u = plsc.bitcast(v, jnp.int32)
    u = (u + 0x7FFF + ((u >> 16) & 1)) & jnp.int32(-65536)
    return plsc.bitcast(u, jnp.float32)


def _sc_body(half, x_hbm, y_hbm, z_hbm, xn2_hbm, cx_hbm, cy_hbm, cz_hbm,
             cn2_hbm, ftab_hbm, g1_hbm, g2_hbm, g3_hbm,
             xv, yv, zv, nv, cxv, cyv, czv, c2v, jb1, jb2, jb3,
             ib1, ib2, ib3, r1, r2, r3, semg0, semg1, sems0, sems1):
    wid = lax.axis_index("s") * 2 + lax.axis_index("c")
    b = wid // 4
    slot = wid % 4
    _SW = _S // 2 // 4
    s0 = half * (_S // 2) + slot * _SW
    o0 = slot * _SW

    pltpu.sync_copy(x_hbm.at[b], xv)
    pltpu.sync_copy(y_hbm.at[b], yv)
    pltpu.sync_copy(z_hbm.at[b], zv)
    pltpu.sync_copy(xn2_hbm.at[b], nv)
    pltpu.sync_copy(cx_hbm.at[b, pl.ds(s0, _SW)], cxv)
    pltpu.sync_copy(cy_hbm.at[b, pl.ds(s0, _SW)], cyv)
    pltpu.sync_copy(cz_hbm.at[b, pl.ds(s0, _SW)], czv)
    pltpu.sync_copy(cn2_hbm.at[b, pl.ds(s0, _SW)], c2v)

    def round_buf(j, _):
        xv[pl.ds(j * 16, 16)] = -2.0 * _bf16_rne(xv[pl.ds(j * 16, 16)])
        yv[pl.ds(j * 16, 16)] = -2.0 * _bf16_rne(yv[pl.ds(j * 16, 16)])
        zv[pl.ds(j * 16, 16)] = -2.0 * _bf16_rne(zv[pl.ds(j * 16, 16)])
        return _

    lax.fori_loop(0, _N // 16, round_buf, 0)

    iota16 = lax.broadcasted_iota(jnp.int32, (16,), 0)
    rsq = tuple(jnp.float32(r * r) for r in _RADII)
    kvec = tuple(jnp.full((16,), k, jnp.int32) for k in _KS)
    zvec = jnp.zeros((16,), jnp.int32)

    def scan_group(grp, ibs):
        def per_centroid(gi, carry2):
            si = grp * _GRP + gi
            t = (si // 16) * 16
            lm = iota16 == si % 16
            cxs = jnp.sum(jnp.where(lm, _bf16_rne(cxv[pl.ds(t, 16)]), 0.0))
            cys = jnp.sum(jnp.where(lm, _bf16_rne(cyv[pl.ds(t, 16)]), 0.0))
            czs = jnp.sum(jnp.where(lm, _bf16_rne(czv[pl.ds(t, 16)]), 0.0))
            c2s = jnp.sum(jnp.where(lm, c2v[pl.ds(t, 16)], 0.0))

            def chunk4(j4, cnts):
                c1v_, c2v_, c3v_ = cnts
                for u in range(4):
                    base = j4 * 64 + u * 16
                    xc = xv[pl.ds(base, 16)]
                    yc = yv[pl.ds(base, 16)]
                    zc = zv[pl.ds(base, 16)]
                    nc = nv[pl.ds(base, 16)]
                    dist = (c2s + nc) + (cxs * xc + cys * yc + czs * zc)
                    idxv = base + iota16
                    m1 = dist <= rsq[0]
                    m2 = dist <= rsq[1]
                    m3 = dist <= rsq[2]
                    plsc.store_compressed(jb1.at[pl.ds(c1v_[0], 16)], idxv,
                                          mask=m1)
                    plsc.store_compressed(jb2.at[pl.ds(c2v_[0], 16)], idxv,
                                          mask=m2)
                    plsc.store_compressed(jb3.at[pl.ds(c3v_[0], 16)], idxv,
                                          mask=m3)
                    c1v_ = jnp.minimum(
                        c1v_ + plsc.all_reduce_population_count(m1), kvec[0])
                    c2v_ = jnp.minimum(
                        c2v_ + plsc.all_reduce_population_count(m2), kvec[1])
                    c3v_ = jnp.minimum(
                        c3v_ + plsc.all_reduce_population_count(m3), kvec[2])
                return c1v_, c2v_, c3v_

            c1v_, c2v_, c3v_ = lax.fori_loop(0, _N // 64, chunk4,
                                             (zvec, zvec, zvec))
            off = b * _N

            def finalize(jb, ib, K, cnt):
                v0 = jb[pl.ds(0, 16)]
                first = jnp.where(cnt > 0, v0[0], _N - 1)
                for t in range(K // 16):
                    v = jb[pl.ds(16 * t, 16)] if t else v0
                    m = (16 * t + iota16) < cnt
                    ib[pl.ds(gi * K + 16 * t, 16)] = (
                        jnp.where(m, v, first) + off)

            finalize(jb1, ibs[0], _KS[0], c1v_[0])
            finalize(jb2, ibs[1], _KS[1], c2v_[0])
            finalize(jb3, ibs[2], _KS[2], c3v_[0])
            return carry2

        lax.fori_loop(0, _GRP, per_centroid, 0)

    ibs = ((ib1.at[0], ib2.at[0], ib3.at[0]),
           (ib1.at[1], ib2.at[1], ib3.at[1]))
    rws = ((r1.at[0], r2.at[0], r3.at[0]),
           (r1.at[1], r2.at[1], r3.at[1]))
    semg = (semg0, semg1)
    sems = (sems0, sems1)
    ghbm = (g1_hbm, g2_hbm, g3_hbm)

    def gather_descs(par):
        return [pltpu.make_async_copy(ftab_hbm.at[ibs[par][k]],
                                      rws[par][k], semg[par])
                for k in range(3)]

    def store_descs(par, grp):
        gs0 = b * (_S // 2) + o0 + grp * _GRP
        return [pltpu.make_async_copy(
            rws[par][k],
            ghbm[k].at[pl.ds(gs0 * _KS[k], _GRP * _KS[k])], sems[par])
            for k in range(3)]

    def issue(descs):
        for d in descs:
            d.start()

    def wait(descs):
        for d in descs:
            d.wait()

    def slot(grp, par, guarded):
        scan_group(grp, ibs[par])
        other = 1 - par
        if guarded:
            wait(gather_descs(other))
            issue(store_descs(other, grp - 1))
            pl.when(grp >= 2)(lambda: wait(store_descs(par, grp - 2)))
        issue(gather_descs(par))

    def pair(g2, carry):
        grp_e = 2 * g2

        @pl.when(g2 == 0)
        def _():
            slot(0, 0, False)

        @pl.when(g2 > 0)
        def _():
            slot(grp_e, 0, True)

        slot(grp_e + 1, 1, True)
        return carry

    npair = _SW // _GRP // 2
    lax.fori_loop(0, npair, pair, 0)
    last = _SW // _GRP - 1
    wait(gather_descs(1))
    issue(store_descs(1, last))
    wait(store_descs(0, last - 1))
    wait(store_descs(1, last))


def _sc_group_gather(x, y, z, xn2, cx, cy, cz, cn2, ftab, half):
    f32 = jnp.float32
    i32 = jnp.int32
    _SW = _S // 2 // 4
    mesh = plsc.VectorSubcoreMesh(core_axis_name="c", subcore_axis_name="s")
    fn = pl.kernel(
        functools.partial(_sc_body, half),
        out_type=(
            jax.ShapeDtypeStruct((_B * (_S // 2) * _KS[0], _FPAD),
                                 jnp.bfloat16),
            jax.ShapeDtypeStruct((_B * (_S // 2) * _KS[1], _FPAD),
                                 jnp.bfloat16),
            jax.ShapeDtypeStruct((_B * (_S // 2) * _KS[2], _FPAD),
                                 jnp.bfloat16),
        ),
        mesh=mesh,
        compiler_params=pltpu.CompilerParams(
            needs_layout_passes=False, use_tc_tiling_on_sc=False),
        scratch_types=[
            pltpu.VMEM((_N,), f32),
            pltpu.VMEM((_N,), f32),
            pltpu.VMEM((_N,), f32),
            pltpu.VMEM((_N,), f32),
            pltpu.VMEM((_SW,), f32),
            pltpu.VMEM((_SW,), f32),
            pltpu.VMEM((_SW,), f32),
            pltpu.VMEM((_SW,), f32),
            pltpu.VMEM((_KS[0] + 16,), i32),
            pltpu.VMEM((_KS[1] + 16,), i32),
            pltpu.VMEM((_KS[2] + 16,), i32),
            pltpu.VMEM((2, _GRP * _KS[0]), i32),
            pltpu.VMEM((2, _GRP * _KS[1]), i32),
            pltpu.VMEM((2, _GRP * _KS[2]), i32),
            pltpu.VMEM((2, _GRP * _KS[0], _FPAD), jnp.bfloat16),
            pltpu.VMEM((2, _GRP * _KS[1], _FPAD), jnp.bfloat16),
            pltpu.VMEM((2, _GRP * _KS[2], _FPAD), jnp.bfloat16),
            pltpu.SemaphoreType.DMA,
            pltpu.SemaphoreType.DMA,
            pltpu.SemaphoreType.DMA,
            pltpu.SemaphoreType.DMA,
        ],
    )
    return fn(x, y, z, xn2, cx, cy, cz, cn2, ftab)


def _mlp_body(K, g_ref, c_ref, w1_ref, b1_ref, wc_ref, w2_ref, b2_ref,
              w3_ref, b3_ref, o_ref):
    bf16 = jnp.bfloat16
    X = g_ref[...]
    H = jnp.dot(X, w1_ref[...].astype(bf16),
                preferred_element_type=jnp.float32)
    O = jnp.dot(c_ref[...].astype(bf16), wc_ref[...].astype(bf16),
                preferred_element_type=jnp.float32)
    SB, C1 = O.shape
    H = H.reshape(SB, K, C1) - O[:, None, :]
    H = jnp.maximum(H + b1_ref[...][None, :, :], 0.0)
    H = H.reshape(SB * K, C1)
    H = jnp.maximum(
        jnp.dot(H.astype(bf16), w2_ref[...].astype(bf16),
                preferred_element_type=jnp.float32)
        + b2_ref[...], 0.0)
    H = jnp.maximum(
        jnp.dot(H.astype(bf16), w3_ref[...].astype(bf16),
                preferred_element_type=jnp.float32)
        + b3_ref[...], 0.0)
    C3 = H.shape[1]
    o_ref[...] = jnp.max(H.reshape(SB, K, C3), axis=1)


def _fold(branch):
    out = []
    for (W, b, gamma, beta) in branch:
        scale = gamma / jnp.sqrt(1.0 + 1e-3)
        out.append((W * scale[None, :], b * scale + beta))
    return out


def _mlp(K, g, c4, folded):
    (W1, b1), (W2, b2), (W3, b3) = folded
    C1, C2, C3 = W1.shape[1], W2.shape[1], W3.shape[1]
    W1p = jnp.zeros((_FPAD, C1), jnp.float32).at[:19, :].set(W1)
    Wc = jnp.zeros((4, C1), jnp.float32).at[:3, :].set(W1[16:19, :])
    SB = 128
    BS = g.shape[0] // K
    grid = (BS // SB,)
    return pl.pallas_call(
        functools.partial(_mlp_body, K),
        grid=grid,
        in_specs=[
            pl.BlockSpec((SB * K, _FPAD), lambda i: (i, 0)),
            pl.BlockSpec((SB, 4), lambda i: (i, 0)),
            pl.BlockSpec((_FPAD, C1), lambda i: (0, 0)),
            pl.BlockSpec((1, C1), lambda i: (0, 0)),
            pl.BlockSpec((4, C1), lambda i: (0, 0)),
            pl.BlockSpec((C1, C2), lambda i: (0, 0)),
            pl.BlockSpec((1, C2), lambda i: (0, 0)),
            pl.BlockSpec((C2, C3), lambda i: (0, 0)),
            pl.BlockSpec((1, C3), lambda i: (0, 0)),
        ],
        out_specs=pl.BlockSpec((SB, C3), lambda i: (i, 0)),
        out_shape=jax.ShapeDtypeStruct((BS, C3), jnp.float32),
    )(g, c4, W1p, b1[None, :], Wc, W2, b2[None, :], W3, b3[None, :])


def kernel(xyz, points, params):
    B, _, N = xyz.shape
    x = xyz[:, 0, :]
    y = xyz[:, 1, :]
    z = xyz[:, 2, :]
    cx, cy, cz, cn2, xn2 = _fps(x, y, z)

    pts_t = jnp.transpose(points, (0, 2, 1))
    xyz_t = jnp.transpose(xyz, (0, 2, 1))
    ftab = jnp.concatenate(
        [pts_t, xyz_t, jnp.zeros((B, N, _FPAD - 19), jnp.float32)],
        axis=-1).reshape(B * N, _FPAD).astype(jnp.bfloat16)

    folded = [_fold(p) for p in params]
    halves = []
    for half in (0, 1):
        g1, g2, g3 = _sc_group_gather(x, y, z, xn2, cx, cy, cz, cn2, ftab,
                                      half)
        sl = slice(half * (_S // 2), (half + 1) * (_S // 2))
        c4 = jnp.stack(
            [cx[:, sl], cy[:, sl], cz[:, sl], jnp.zeros_like(cx[:, sl])],
            axis=-1).reshape(B * (_S // 2), 4)
        outs = [_mlp(_KS[bi], g, c4, folded[bi])
                for bi, g in enumerate((g1, g2, g3))]
        halves.append(
            jnp.concatenate(outs, axis=-1).reshape(B, _S // 2, 320))
    new_points = jnp.concatenate(halves, axis=1)
    new_points = jnp.transpose(new_points, (0, 2, 1))
    new_xyz = jnp.stack([cx, cy, cz], axis=1)
    return new_xyz, new_points
